# Initial kernel scaffold; baseline (speedup 1.0000x reference)
import jax
import jax.numpy as jnp
from jax.experimental import pallas as pl


def kernel(cls_logits_0, cls_logits_1, cls_logits_2, cls_logits_3, cls_logits_4, ctr_logits_0, ctr_logits_1, ctr_logits_2, ctr_logits_3, ctr_logits_4, reg_preds_0, reg_preds_1, reg_preds_2, reg_preds_3, reg_preds_4, gt_boxes, classes):
    raise NotImplementedError("write your pallas kernel here")


# speedup vs baseline: 5.5295x; 5.5295x over previous
import functools

import numpy as np
import jax
import jax.numpy as jnp
from jax import lax
from jax.experimental import pallas as pl
from jax.experimental.pallas import tpu as pltpu
from jax.experimental.pallas import tpu_sc as plsc

_SHAPES = [(128, 128), (64, 64), (32, 32), (16, 16), (8, 8)]
_STRIDES = [8, 16, 32, 64, 128]
_LIMITS = [(-1.0, 64.0), (64.0, 128.0), (128.0, 256.0), (256.0, 512.0), (512.0, 999999.0)]
_BIG = 99999999.0

_B, _M, _MP = 4, 50, 64
_HW = sum(h * w for h, w in _SHAPES)
_NW = 32
_WPB = _NW // _B
_LANES = 16
_PER_W = 2736
_HWP = _WPB * _PER_W
_CHUNKS = _PER_W // _LANES


def _build_loc_table():
    xs, ys, los, his, rads = [], [], [], [], []
    for (h, w), s, (lo, hi) in zip(_SHAPES, _STRIDES, _LIMITS):
        ix = np.arange(h * w)
        xs.append((ix % w).astype(np.float32) * s + s // 2)
        ys.append((ix // w).astype(np.float32) * s + s // 2)
        los.append(np.full(h * w, lo, np.float32))
        his.append(np.full(h * w, hi, np.float32))
        rads.append(np.full(h * w, s * 1.5, np.float32))
    rows = [np.concatenate(a) for a in (xs, ys, los, his, rads)]
    pad = _HWP - _HW
    fill = (-1e6, -1e6, 0.0, -1.0, -1.0)
    rows = [np.concatenate([r, np.full(pad, f, np.float32)]) for r, f in zip(rows, fill)]
    return np.stack(rows)


_LOC_TABLE = _build_loc_table()


def _sc_body(loc_hbm, boxes_hbm, classes_hbm, cls_out, ctr_out, reg_out,
             x_v, y_v, lo_v, hi_v, rad_v, boxes_v, classes_v,
             cls_ov, ctr_ov, reg_ov, tab_v):
    wid = lax.axis_index("s") * 2 + lax.axis_index("c")
    batch = wid // _WPB
    off = (wid % _WPB) * _PER_W

    for i, dst in enumerate((x_v, y_v, lo_v, hi_v, rad_v)):
        pltpu.sync_copy(loc_hbm.at[pl.ds(i * _HWP + off, _PER_W)], dst)
    pltpu.sync_copy(boxes_hbm.at[pl.ds(batch * 4 * _MP, 4 * _MP)], boxes_v)
    pltpu.sync_copy(classes_hbm.at[pl.ds(batch * _MP, _MP)], classes_v)

    for g in range(_MP // _LANES):
        x1v = boxes_v[pl.ds(0 * _MP + g * _LANES, _LANES)]
        y1v = boxes_v[pl.ds(1 * _MP + g * _LANES, _LANES)]
        x2v = boxes_v[pl.ds(2 * _MP + g * _LANES, _LANES)]
        y2v = boxes_v[pl.ds(3 * _MP + g * _LANES, _LANES)]
        cxv = (x1v + x2v) * 0.5
        cyv = (y1v + y2v) * 0.5
        areav = (x2v - x1v) * (y2v - y1v)
        for lane in range(_LANES):
            k = g * _LANES + lane
            if k >= _M:
                break
            for p, src in enumerate((x1v, y1v, x2v, y2v, cxv, cyv, areav)):
                tab_v[p, pl.ds(k * _LANES, _LANES)] = jnp.broadcast_to(
                    src[lane], (_LANES,))

    def chunk(c, carry):
        base = c * _LANES
        xv = x_v[pl.ds(base, _LANES)]
        yv = y_v[pl.ds(base, _LANES)]
        lov = lo_v[pl.ds(base, _LANES)]
        hiv = hi_v[pl.ds(base, _LANES)]
        radv = rad_v[pl.ds(base, _LANES)]

        best_a = jnp.full((_LANES,), _BIG, jnp.float32)
        best_i = jnp.zeros((_LANES,), jnp.int32)
        for k in range(_M):
            ks = pl.ds(k * _LANES, _LANES)
            x1 = tab_v[0, ks]
            y1 = tab_v[1, ks]
            x2 = tab_v[2, ks]
            y2 = tab_v[3, ks]
            cx = tab_v[4, ks]
            cy = tab_v[5, ks]
            area = tab_v[6, ks]
            l = xv - x1
            t = yv - y1
            r = x2 - xv
            b = y2 - yv
            omin = jnp.minimum(jnp.minimum(l, t), jnp.minimum(r, b))
            omax = jnp.maximum(jnp.maximum(l, t), jnp.maximum(r, b))
            m_in = omin > 0.0
            m_lv = (omax > lov) & (omax <= hiv)
            m_c = jnp.maximum(jnp.abs(xv - cx), jnp.abs(yv - cy)) < radv
            mask = m_in & m_lv & m_c
            ak = jnp.where(mask, area, _BIG)
            upd = ak < best_a
            best_a = jnp.where(upd, ak, best_a)
            best_i = jnp.where(upd, k, best_i)

        pos = best_a < _BIG
        x1g = plsc.load_gather(boxes_v, [best_i])
        y1g = plsc.load_gather(boxes_v, [best_i + _MP])
        x2g = plsc.load_gather(boxes_v, [best_i + 2 * _MP])
        y2g = plsc.load_gather(boxes_v, [best_i + 3 * _MP])
        clsg = plsc.load_gather(classes_v, [best_i])
        lg = xv - x1g
        tg = yv - y1g
        rg = x2g - xv
        bg = y2g - yv
        lrmin = jnp.minimum(lg, rg)
        lrmax = jnp.maximum(lg, rg)
        tbmin = jnp.minimum(tg, bg)
        tbmax = jnp.maximum(tg, bg)
        num = jnp.where(pos, lrmin * tbmin, 1.0)
        den = jnp.where(pos, jnp.maximum(lrmax * tbmax + 1e-10, 0.0), 1.0)
        ratio = num / den
        bits = lax.bitcast_convert_type(ratio, jnp.int32)
        sq = lax.bitcast_convert_type(
            lax.shift_right_logical(bits, 1) + 0x1FBD1DF5, jnp.float32)
        for _ in range(3):
            sq = 0.5 * (sq + ratio / sq)

        sl = pl.ds(base, _LANES)
        cls_ov[sl] = jnp.where(pos, clsg, 0)
        ctr_ov[sl] = jnp.where(pos, sq, -1.0)
        reg_ov[0, sl] = jnp.where(pos, lg, -1.0)
        reg_ov[1, sl] = jnp.where(pos, tg, -1.0)
        reg_ov[2, sl] = jnp.where(pos, rg, -1.0)
        reg_ov[3, sl] = jnp.where(pos, bg, -1.0)
        return carry

    lax.fori_loop(0, _CHUNKS, chunk, 0)

    pltpu.sync_copy(cls_ov, cls_out.at[pl.ds(batch * _HWP + off, _PER_W)])
    pltpu.sync_copy(ctr_ov, ctr_out.at[pl.ds(batch * _HWP + off, _PER_W)])
    for j in range(4):
        pltpu.sync_copy(reg_ov.at[j],
                        reg_out.at[pl.ds((batch * 4 + j) * _HWP + off, _PER_W)])


@jax.jit
def _gen_targets(gt_boxes, classes):
    loc = jnp.asarray(_LOC_TABLE).reshape(-1)
    boxes_pl = jnp.transpose(gt_boxes, (0, 2, 1))
    boxes_pl = jnp.pad(boxes_pl, ((0, 0), (0, 0), (0, _MP - _M))).reshape(-1)
    classes_p = jnp.pad(classes, ((0, 0), (0, _MP - _M))).reshape(-1)

    mesh = plsc.VectorSubcoreMesh(core_axis_name="c", subcore_axis_name="s")
    run = functools.partial(
        pl.kernel,
        mesh=mesh,
        compiler_params=pltpu.CompilerParams(
            needs_layout_passes=False, use_tc_tiling_on_sc=False),
        out_type=[
            jax.ShapeDtypeStruct((_B * _HWP,), jnp.int32),
            jax.ShapeDtypeStruct((_B * _HWP,), jnp.float32),
            jax.ShapeDtypeStruct((_B * 4 * _HWP,), jnp.float32),
        ],
        scratch_types=[
            pltpu.VMEM((_PER_W,), jnp.float32),
            pltpu.VMEM((_PER_W,), jnp.float32),
            pltpu.VMEM((_PER_W,), jnp.float32),
            pltpu.VMEM((_PER_W,), jnp.float32),
            pltpu.VMEM((_PER_W,), jnp.float32),
            pltpu.VMEM((4 * _MP,), jnp.float32),
            pltpu.VMEM((_MP,), jnp.int32),
            pltpu.VMEM((_PER_W,), jnp.int32),
            pltpu.VMEM((_PER_W,), jnp.float32),
            pltpu.VMEM((4, _PER_W), jnp.float32),
            pltpu.VMEM((7, _M * _LANES), jnp.float32),
        ],
    )(_sc_body)
    cls_p, ctr_p, reg_p = run(loc, boxes_pl, classes_p)
    cls_t = cls_p.reshape(_B, _HWP)[:, :_HW, None]
    ctr_t = ctr_p.reshape(_B, _HWP)[:, :_HW, None]
    reg_t = jnp.transpose(reg_p.reshape(_B, 4, _HWP), (0, 2, 1))[:, :_HW, :]
    return cls_t, ctr_t, reg_t


def kernel(cls_logits_0, cls_logits_1, cls_logits_2, cls_logits_3, cls_logits_4,
           ctr_logits_0, ctr_logits_1, ctr_logits_2, ctr_logits_3, ctr_logits_4,
           reg_preds_0, reg_preds_1, reg_preds_2, reg_preds_3, reg_preds_4,
           gt_boxes, classes):
    return _gen_targets(gt_boxes, classes)
